# Initial kernel scaffold; baseline (speedup 1.0000x reference)
import jax
import jax.numpy as jnp
from jax.experimental import pallas as pl


def kernel(x, edge_index, W1, b1, g1, be1, W2, b2, g2, be2, Wfc, bfc):
    raise NotImplementedError("write your pallas kernel here")


# speedup vs baseline: 7.3038x; 7.3038x over previous
import functools

import jax
import jax.numpy as jnp
from jax import lax
from jax.experimental import pallas as pl
from jax.experimental.pallas import tpu as pltpu
from jax.experimental.pallas import tpu_sc as plsc

N = 10000
D = 256
H = 512
C = 2
E = 160000
NQ = 4
QW = 128
NT = 16
NC = 2
NP = 10240
BM = 640
GI = NP // BM
BM_F = 1000
GI_F = N // BM_F

EB_N = 125
EB_W = 80
ROWS_PER_TILE = NP // NT

DEG_B_N = 40
DEG_B_W = 128
DEG_PAD = NC * NT * DEG_B_N * DEG_B_W - E
DEG_ROWS = NP + 256
DEG_ZROWS = DEG_ROWS // NT

_mesh = functools.partial(
    plsc.VectorSubcoreMesh, core_axis_name="c", subcore_axis_name="s")


def _deg_body(dstp, zeros_h, ones_h, parts, acc_sh, dst_v, ones_v):
  c = lax.axis_index("c")
  s = lax.axis_index("s")
  w = c * NT + s
  pltpu.sync_copy(zeros_h.at[pl.ds(s * DEG_ZROWS, DEG_ZROWS)],
                  acc_sh.at[pl.ds(s * DEG_ZROWS, DEG_ZROWS)])
  pltpu.sync_copy(ones_h, ones_v)
  pltpu.sync_copy(dstp.at[w], dst_v)
  plsc.subcore_barrier()

  def step(j, carry):
    pltpu.sync_copy(ones_v, acc_sh.at[dst_v.at[j]], add=True)
    return carry

  lax.fori_loop(0, DEG_B_N, step, 0)
  plsc.subcore_barrier()
  pltpu.sync_copy(acc_sh.at[pl.ds(s * ROWS_PER_TILE, ROWS_PER_TILE)],
                  parts.at[c, pl.ds(s * ROWS_PER_TILE, ROWS_PER_TILE)])


def _deg_call(dstp, zeros_h, ones_h):
  f = pl.kernel(
      _deg_body,
      out_type=jax.ShapeDtypeStruct((NC, NP, 16), jnp.float32),
      mesh=_mesh(),
      scratch_types=[
          pltpu.VMEM_SHARED((DEG_ROWS, 16), jnp.float32),
          pltpu.VMEM((DEG_B_N, DEG_B_W), jnp.int32),
          pltpu.VMEM((DEG_B_W, 16), jnp.float32),
      ],
  )
  return f(dstp, zeros_h, ones_h)


def _agg_body(hpq, src_r, dst_r, aggq, acc_sh, src_v, dst_v, rows_a, sem_a):
  c = lax.axis_index("c")
  s = lax.axis_index("s")
  pltpu.sync_copy(src_r.at[s], src_v)
  pltpu.sync_copy(dst_r.at[s], dst_v)
  for qq in range(NQ // NC):
    q = c * (NQ // NC) + qq
    hq = hpq.at[q]
    pltpu.sync_copy(hq.at[pl.ds(s * ROWS_PER_TILE, ROWS_PER_TILE)],
                    acc_sh.at[pl.ds(s * ROWS_PER_TILE, ROWS_PER_TILE)])
    plsc.subcore_barrier()

    def step(j, carry):
      pltpu.async_copy(hq.at[src_v.at[j]], rows_a, sem_a).wait()
      pltpu.sync_copy(rows_a, acc_sh.at[dst_v.at[j]], add=True)
      return carry

    lax.fori_loop(0, EB_N, step, 0)
    plsc.subcore_barrier()
    pltpu.sync_copy(acc_sh.at[pl.ds(s * ROWS_PER_TILE, ROWS_PER_TILE)],
                    aggq.at[q, pl.ds(s * ROWS_PER_TILE, ROWS_PER_TILE)])
    plsc.subcore_barrier()


def _agg_call(hpq, src_r, dst_r):
  f = pl.kernel(
      _agg_body,
      out_type=jax.ShapeDtypeStruct((NQ, NP, QW), jnp.float32),
      mesh=_mesh(),
      scratch_types=[
          pltpu.VMEM_SHARED((NP, QW), jnp.float32),
          pltpu.VMEM((EB_N, EB_W), jnp.int32),
          pltpu.VMEM((EB_N, EB_W), jnp.int32),
          pltpu.VMEM((EB_W, QW), jnp.float32),
          pltpu.SemaphoreType.DMA,
      ],
  )
  return f(hpq, src_r, dst_r)


_PREC = lax.Precision.HIGHEST


def _dinv_body(parts, dinv):
  i = pl.program_id(0)
  d = parts[0, :, 0:1] + parts[1, :, 0:1] + 1.0
  row = i * BM + lax.broadcasted_iota(jnp.int32, (BM, 1), 0)
  dinv[...] = jnp.where(row < N, lax.rsqrt(d), 0.0)


def _dinv_call(parts):
  return pl.pallas_call(
      _dinv_body,
      grid=(GI,),
      in_specs=[pl.BlockSpec((NC, BM, 16), lambda i: (0, i, 0))],
      out_specs=pl.BlockSpec((BM, 1), lambda i: (i, 0)),
      out_shape=jax.ShapeDtypeStruct((NP, 1), jnp.float32),
  )(parts)


def _hp_body(x, w, dinv, out):
  out[0] = jnp.dot(x[...], w[...], preferred_element_type=jnp.float32,
                   precision=_PREC) * dinv[...]


def _hp_call(x, w, dinv):
  din = x.shape[1]
  return pl.pallas_call(
      _hp_body,
      grid=(GI, NQ),
      in_specs=[
          pl.BlockSpec((BM, din), lambda i, q: (i, 0)),
          pl.BlockSpec((din, QW), lambda i, q: (0, q)),
          pl.BlockSpec((BM, 1), lambda i, q: (i, 0)),
      ],
      out_specs=pl.BlockSpec((1, BM, QW), lambda i, q: (q, i, 0)),
      out_shape=jax.ShapeDtypeStruct((NQ, NP, QW), jnp.float32),
  )(x, w, dinv)


def _stats_body(aggq, dinv, t, sref, ssref):
  i = pl.program_id(1)
  tv = aggq[0] * dinv[...]
  t[0] = tv
  col = jnp.sum(tv, axis=0, keepdims=True)
  col2 = jnp.sum(tv * tv, axis=0, keepdims=True)

  @pl.when(i == 0)
  def _():
    sref[0] = col
    ssref[0] = col2

  @pl.when(i != 0)
  def _():
    sref[0] += col
    ssref[0] += col2


def _stats_call(aggq, dinv):
  return pl.pallas_call(
      _stats_body,
      grid=(NQ, GI),
      in_specs=[
          pl.BlockSpec((1, BM, QW), lambda q, i: (q, i, 0)),
          pl.BlockSpec((BM, 1), lambda q, i: (i, 0)),
      ],
      out_specs=[
          pl.BlockSpec((1, BM, QW), lambda q, i: (q, i, 0)),
          pl.BlockSpec((1, 1, QW), lambda q, i: (q, 0, 0)),
          pl.BlockSpec((1, 1, QW), lambda q, i: (q, 0, 0)),
      ],
      out_shape=[
          jax.ShapeDtypeStruct((NQ, NP, QW), jnp.float32),
          jax.ShapeDtypeStruct((NQ, 1, QW), jnp.float32),
          jax.ShapeDtypeStruct((NQ, 1, QW), jnp.float32),
      ],
  )(aggq, dinv)


def _bn_relu(t, s, ss, g, be):
  m = s / N
  v = ss / N - m * m
  return jnp.maximum((t - m) * lax.rsqrt(v + 1e-5) * g + be, 0.0)


def _mid_body(t1, s1, ss1, g1, be1, w2, dinv, out):
  acc = jnp.zeros((BM, QW), jnp.float32)
  for qi in range(NQ):
    h = _bn_relu(t1[qi], s1[qi], ss1[qi], g1[qi], be1[qi])
    acc += jnp.dot(h, w2[qi * QW:(qi + 1) * QW, :],
                   preferred_element_type=jnp.float32, precision=_PREC)
  out[0] = acc * dinv[...]


def _mid_call(t1, s1, ss1, g1, be1, w2, dinv):
  return pl.pallas_call(
      _mid_body,
      grid=(GI, NQ),
      in_specs=[
          pl.BlockSpec((NQ, BM, QW), lambda i, q: (0, i, 0)),
          pl.BlockSpec((NQ, 1, QW), lambda i, q: (0, 0, 0)),
          pl.BlockSpec((NQ, 1, QW), lambda i, q: (0, 0, 0)),
          pl.BlockSpec((NQ, QW), lambda i, q: (0, 0)),
          pl.BlockSpec((NQ, QW), lambda i, q: (0, 0)),
          pl.BlockSpec((H, QW), lambda i, q: (0, q)),
          pl.BlockSpec((BM, 1), lambda i, q: (i, 0)),
      ],
      out_specs=pl.BlockSpec((1, BM, QW), lambda i, q: (q, i, 0)),
      out_shape=jax.ShapeDtypeStruct((NQ, NP, QW), jnp.float32),
  )(t1, s1, ss1, g1, be1, w2, dinv)


def _fin_body(t2, s2, ss2, g2, be2, wfc, bfc, out):
  acc = jnp.zeros((BM_F, C), jnp.float32)
  for qi in range(NQ):
    h = _bn_relu(t2[qi], s2[qi], ss2[qi], g2[qi], be2[qi])
    acc += jnp.dot(h, wfc[qi], preferred_element_type=jnp.float32,
                   precision=_PREC)
  out[...] = acc + bfc[...]


def _fin_call(t2, s2, ss2, g2, be2, wfcq, bfc):
  return pl.pallas_call(
      _fin_body,
      grid=(GI_F,),
      in_specs=[
          pl.BlockSpec((NQ, BM_F, QW), lambda i: (0, i, 0)),
          pl.BlockSpec((NQ, 1, QW), lambda i: (0, 0, 0)),
          pl.BlockSpec((NQ, 1, QW), lambda i: (0, 0, 0)),
          pl.BlockSpec((NQ, QW), lambda i: (0, 0)),
          pl.BlockSpec((NQ, QW), lambda i: (0, 0)),
          pl.BlockSpec((NQ, QW, C), lambda i: (0, 0, 0)),
          pl.BlockSpec((1, C), lambda i: (0, 0)),
      ],
      out_specs=pl.BlockSpec((BM_F, C), lambda i: (i, 0)),
      out_shape=jax.ShapeDtypeStruct((N, C), jnp.float32),
  )(t2, s2, ss2, g2, be2, wfcq, bfc)


def kernel(x, edge_index, W1, b1, g1, be1, W2, b2, g2, be2, Wfc, bfc):
  src = edge_index[0]
  dst = edge_index[1]
  src_r = src.reshape(NT, EB_N, EB_W)
  dst_r = dst.reshape(NT, EB_N, EB_W)
  pad = NP + (jnp.arange(DEG_PAD, dtype=jnp.int32) % 256)
  dstp = jnp.concatenate([dst, pad]).reshape(NC * NT, DEG_B_N, DEG_B_W)
  zeros_h = jnp.zeros((DEG_ROWS, 16), jnp.float32)
  ones_h = jnp.ones((DEG_B_W, 16), jnp.float32)
  xp = jnp.pad(x, ((0, NP - N), (0, 0)))

  parts = _deg_call(dstp, zeros_h, ones_h)
  dinv = _dinv_call(parts)

  g1q = g1.reshape(NQ, QW)
  be1q = be1.reshape(NQ, QW)
  g2q = g2.reshape(NQ, QW)
  be2q = be2.reshape(NQ, QW)
  wfcq = Wfc.reshape(NQ, QW, C)
  bfc2 = bfc.reshape(1, C)

  hp1 = _hp_call(xp, W1, dinv)
  agg1 = _agg_call(hp1, src_r, dst_r)
  t1, s1, ss1 = _stats_call(agg1, dinv)
  hp2 = _mid_call(t1, s1, ss1, g1q, be1q, W2, dinv)
  agg2 = _agg_call(hp2, src_r, dst_r)
  t2, s2, ss2 = _stats_call(agg2, dinv)
  out = _fin_call(t2, s2, ss2, g2q, be2q, wfcq, bfc2)
  return out


# speedup vs baseline: 7.9351x; 1.0864x over previous
import functools

import jax
import jax.numpy as jnp
from jax import lax
from jax.experimental import pallas as pl
from jax.experimental.pallas import tpu as pltpu
from jax.experimental.pallas import tpu_sc as plsc

N = 10000
D = 256
H = 512
C = 2
E = 160000
NQ = 4
QW = 128
NT = 16
NC = 2
NP = 10240
BM = 640
GI = NP // BM
BM_F = 1000
GI_F = N // BM_F

EB_W = 80
CB = 8
NCH = 16
ET_PAD = NCH * CB * EB_W
EPAD = NT * ET_PAD - E
ROWS_PER_TILE = NP // NT

DEG_B_N = 40
DEG_B_W = 128
DEG_PAD = NC * NT * DEG_B_N * DEG_B_W - E
DEG_ROWS = NP + 256
DEG_ZROWS = DEG_ROWS // NT

_mesh = functools.partial(
    plsc.VectorSubcoreMesh, core_axis_name="c", subcore_axis_name="s")


def _deg_body(dstp, zeros_h, ones_h, parts, acc_sh, dst_v, ones_v):
  c = lax.axis_index("c")
  s = lax.axis_index("s")
  w = c * NT + s
  pltpu.sync_copy(zeros_h.at[pl.ds(s * DEG_ZROWS, DEG_ZROWS)],
                  acc_sh.at[pl.ds(s * DEG_ZROWS, DEG_ZROWS)])
  pltpu.sync_copy(ones_h, ones_v)
  pltpu.sync_copy(dstp.at[w], dst_v)
  plsc.subcore_barrier()

  def step(j, carry):
    pltpu.sync_copy(ones_v, acc_sh.at[dst_v.at[j]], add=True)
    return carry

  lax.fori_loop(0, DEG_B_N, step, 0)
  plsc.subcore_barrier()
  pltpu.sync_copy(acc_sh.at[pl.ds(s * ROWS_PER_TILE, ROWS_PER_TILE)],
                  parts.at[c, pl.ds(s * ROWS_PER_TILE, ROWS_PER_TILE)])


def _deg_call(dstp, zeros_h, ones_h):
  f = pl.kernel(
      _deg_body,
      out_type=jax.ShapeDtypeStruct((NC, NP, 16), jnp.float32),
      mesh=_mesh(),
      scratch_types=[
          pltpu.VMEM_SHARED((DEG_ROWS, 16), jnp.float32),
          pltpu.VMEM((DEG_B_N, DEG_B_W), jnp.int32),
          pltpu.VMEM((DEG_B_W, 16), jnp.float32),
      ],
  )
  return f(dstp, zeros_h, ones_h)


def _agg_body(hpq, src_r, dst_r, aggq, acc_sh, src_v, dst_v, rows_0, rows_1,
              sem_g0, sem_g1):
  c = lax.axis_index("c")
  s = lax.axis_index("s")
  rows = (rows_0, rows_1)
  gsems = (sem_g0, sem_g1)

  for qq in range(NQ // NC):
    q = c * (NQ // NC) + qq
    hq = hpq.at[q]
    pltpu.sync_copy(hq.at[pl.ds(s * ROWS_PER_TILE, ROWS_PER_TILE)],
                    acc_sh.at[pl.ds(s * ROWS_PER_TILE, ROWS_PER_TILE)])
    plsc.subcore_barrier()

    def chunk(ch, carry):
      pltpu.sync_copy(src_r.at[s, pl.ds(ch * CB, CB)], src_v)
      pltpu.sync_copy(dst_r.at[s, pl.ds(ch * CB, CB)], dst_v)
      pend = pltpu.async_copy(hq.at[src_v.at[0]], rows[0], gsems[0])
      for b in range(CB):
        pend.wait()
        if b + 1 < CB:
          pend = pltpu.async_copy(hq.at[src_v.at[b + 1]], rows[(b + 1) % 2],
                                  gsems[(b + 1) % 2])
        pltpu.sync_copy(rows[b % 2], acc_sh.at[dst_v.at[b]], add=True)
      return carry

    lax.fori_loop(0, NCH, chunk, 0)
    plsc.subcore_barrier()
    pltpu.sync_copy(acc_sh.at[pl.ds(s * ROWS_PER_TILE, ROWS_PER_TILE)],
                    aggq.at[q, pl.ds(s * ROWS_PER_TILE, ROWS_PER_TILE)])
    plsc.subcore_barrier()


def _agg_call(hpq, src_r, dst_r):
  f = pl.kernel(
      _agg_body,
      out_type=jax.ShapeDtypeStruct((NQ, NP, QW), jnp.float32),
      mesh=_mesh(),
      scratch_types=[
          pltpu.VMEM_SHARED((NP, QW), jnp.float32),
          pltpu.VMEM((CB, EB_W), jnp.int32),
          pltpu.VMEM((CB, EB_W), jnp.int32),
          pltpu.VMEM((EB_W, QW), jnp.float32),
          pltpu.VMEM((EB_W, QW), jnp.float32),
          pltpu.SemaphoreType.DMA,
          pltpu.SemaphoreType.DMA,
      ],
  )
  return f(hpq, src_r, dst_r)


_PREC = lax.Precision.HIGHEST


def _dinv_body(parts, dinv):
  i = pl.program_id(0)
  d = parts[0, :, 0:1] + parts[1, :, 0:1] + 1.0
  row = i * BM + lax.broadcasted_iota(jnp.int32, (BM, 1), 0)
  dinv[...] = jnp.where(row < N, lax.rsqrt(d), 0.0)


def _dinv_call(parts):
  return pl.pallas_call(
      _dinv_body,
      grid=(GI,),
      in_specs=[pl.BlockSpec((NC, BM, 16), lambda i: (0, i, 0))],
      out_specs=pl.BlockSpec((BM, 1), lambda i: (i, 0)),
      out_shape=jax.ShapeDtypeStruct((NP, 1), jnp.float32),
  )(parts)


def _hp_body(x, w, dinv, out):
  out[0] = jnp.dot(x[...], w[...], preferred_element_type=jnp.float32,
                   precision=_PREC) * dinv[...]


def _hp_call(x, w, dinv):
  din = x.shape[1]
  return pl.pallas_call(
      _hp_body,
      grid=(GI, NQ),
      in_specs=[
          pl.BlockSpec((BM, din), lambda i, q: (i, 0)),
          pl.BlockSpec((din, QW), lambda i, q: (0, q)),
          pl.BlockSpec((BM, 1), lambda i, q: (i, 0)),
      ],
      out_specs=pl.BlockSpec((1, BM, QW), lambda i, q: (q, i, 0)),
      out_shape=jax.ShapeDtypeStruct((NQ, NP, QW), jnp.float32),
  )(x, w, dinv)


def _stats_body(aggq, dinv, t, sref, ssref):
  i = pl.program_id(1)
  tv = aggq[0] * dinv[...]
  t[0] = tv
  col = jnp.sum(tv, axis=0, keepdims=True)
  col2 = jnp.sum(tv * tv, axis=0, keepdims=True)

  @pl.when(i == 0)
  def _():
    sref[0] = col
    ssref[0] = col2

  @pl.when(i != 0)
  def _():
    sref[0] += col
    ssref[0] += col2


def _stats_call(aggq, dinv):
  return pl.pallas_call(
      _stats_body,
      grid=(NQ, GI),
      in_specs=[
          pl.BlockSpec((1, BM, QW), lambda q, i: (q, i, 0)),
          pl.BlockSpec((BM, 1), lambda q, i: (i, 0)),
      ],
      out_specs=[
          pl.BlockSpec((1, BM, QW), lambda q, i: (q, i, 0)),
          pl.BlockSpec((1, 1, QW), lambda q, i: (q, 0, 0)),
          pl.BlockSpec((1, 1, QW), lambda q, i: (q, 0, 0)),
      ],
      out_shape=[
          jax.ShapeDtypeStruct((NQ, NP, QW), jnp.float32),
          jax.ShapeDtypeStruct((NQ, 1, QW), jnp.float32),
          jax.ShapeDtypeStruct((NQ, 1, QW), jnp.float32),
      ],
  )(aggq, dinv)


def _bn_relu(t, s, ss, g, be):
  m = s / N
  v = ss / N - m * m
  return jnp.maximum((t - m) * lax.rsqrt(v + 1e-5) * g + be, 0.0)


def _mid_body(t1, s1, ss1, g1, be1, w2, dinv, out):
  acc = jnp.zeros((BM, QW), jnp.float32)
  for qi in range(NQ):
    h = _bn_relu(t1[qi], s1[qi], ss1[qi], g1[qi], be1[qi])
    acc += jnp.dot(h, w2[qi * QW:(qi + 1) * QW, :],
                   preferred_element_type=jnp.float32, precision=_PREC)
  out[0] = acc * dinv[...]


def _mid_call(t1, s1, ss1, g1, be1, w2, dinv):
  return pl.pallas_call(
      _mid_body,
      grid=(GI, NQ),
      in_specs=[
          pl.BlockSpec((NQ, BM, QW), lambda i, q: (0, i, 0)),
          pl.BlockSpec((NQ, 1, QW), lambda i, q: (0, 0, 0)),
          pl.BlockSpec((NQ, 1, QW), lambda i, q: (0, 0, 0)),
          pl.BlockSpec((NQ, QW), lambda i, q: (0, 0)),
          pl.BlockSpec((NQ, QW), lambda i, q: (0, 0)),
          pl.BlockSpec((H, QW), lambda i, q: (0, q)),
          pl.BlockSpec((BM, 1), lambda i, q: (i, 0)),
      ],
      out_specs=pl.BlockSpec((1, BM, QW), lambda i, q: (q, i, 0)),
      out_shape=jax.ShapeDtypeStruct((NQ, NP, QW), jnp.float32),
  )(t1, s1, ss1, g1, be1, w2, dinv)


def _fin_body(t2, s2, ss2, g2, be2, wfc, bfc, out):
  acc = jnp.zeros((BM_F, C), jnp.float32)
  for qi in range(NQ):
    h = _bn_relu(t2[qi], s2[qi], ss2[qi], g2[qi], be2[qi])
    acc += jnp.dot(h, wfc[qi], preferred_element_type=jnp.float32,
                   precision=_PREC)
  out[...] = acc + bfc[...]


def _fin_call(t2, s2, ss2, g2, be2, wfcq, bfc):
  return pl.pallas_call(
      _fin_body,
      grid=(GI_F,),
      in_specs=[
          pl.BlockSpec((NQ, BM_F, QW), lambda i: (0, i, 0)),
          pl.BlockSpec((NQ, 1, QW), lambda i: (0, 0, 0)),
          pl.BlockSpec((NQ, 1, QW), lambda i: (0, 0, 0)),
          pl.BlockSpec((NQ, QW), lambda i: (0, 0)),
          pl.BlockSpec((NQ, QW), lambda i: (0, 0)),
          pl.BlockSpec((NQ, QW, C), lambda i: (0, 0, 0)),
          pl.BlockSpec((1, C), lambda i: (0, 0)),
      ],
      out_specs=pl.BlockSpec((BM_F, C), lambda i: (i, 0)),
      out_shape=jax.ShapeDtypeStruct((N, C), jnp.float32),
  )(t2, s2, ss2, g2, be2, wfcq, bfc)


def kernel(x, edge_index, W1, b1, g1, be1, W2, b2, g2, be2, Wfc, bfc):
  src = edge_index[0]
  dst = edge_index[1]
  tpad = ET_PAD - E // NT
  tid = jnp.arange(NT, dtype=jnp.int32)[:, None]
  k = jnp.arange(tpad, dtype=jnp.int32)[None, :]
  pad_src = (tid * 977 + k * 41) % N
  pad_dst = N + (tid * 31 + k) % (NP - N)
  src_r = jnp.concatenate([src.reshape(NT, E // NT), pad_src],
                          axis=1).reshape(NT, NCH * CB, EB_W)
  dst_r = jnp.concatenate([dst.reshape(NT, E // NT), pad_dst],
                          axis=1).reshape(NT, NCH * CB, EB_W)
  pad = NP + (jnp.arange(DEG_PAD, dtype=jnp.int32) % 256)
  dstp = jnp.concatenate([dst, pad]).reshape(NC * NT, DEG_B_N, DEG_B_W)
  zeros_h = jnp.zeros((DEG_ROWS, 16), jnp.float32)
  ones_h = jnp.ones((DEG_B_W, 16), jnp.float32)
  xp = jnp.pad(x, ((0, NP - N), (0, 0)))

  parts = _deg_call(dstp, zeros_h, ones_h)
  dinv = _dinv_call(parts)

  g1q = g1.reshape(NQ, QW)
  be1q = be1.reshape(NQ, QW)
  g2q = g2.reshape(NQ, QW)
  be2q = be2.reshape(NQ, QW)
  wfcq = Wfc.reshape(NQ, QW, C)
  bfc2 = bfc.reshape(1, C)

  hp1 = _hp_call(xp, W1, dinv)
  agg1 = _agg_call(hp1, src_r, dst_r)
  t1, s1, ss1 = _stats_call(agg1, dinv)
  hp2 = _mid_call(t1, s1, ss1, g1q, be1q, W2, dinv)
  agg2 = _agg_call(hp2, src_r, dst_r)
  t2, s2, ss2 = _stats_call(agg2, dinv)
  out = _fin_call(t2, s2, ss2, g2q, be2q, wfcq, bfc2)
  return out


# speedup vs baseline: 8.5947x; 1.0831x over previous
import functools

import jax
import jax.numpy as jnp
from jax import lax
from jax.experimental import pallas as pl
from jax.experimental.pallas import tpu as pltpu
from jax.experimental.pallas import tpu_sc as plsc

N = 10000
D = 256
H = 512
C = 2
E = 160000
NQ = 4
QW = 128
NT = 16
NC = 2
NP = 10240
BM = 640
GI = NP // BM
BM_F = 1000
GI_F = N // BM_F

EB_W = 80
CB = 8
NCH = 16
ET_PAD = NCH * CB * EB_W
EPAD = NT * ET_PAD - E
ROWS_PER_TILE = NP // NT

DEG_B_N = 40
DEG_B_W = 128
DEG_PAD = NC * NT * DEG_B_N * DEG_B_W - E
DEG_ROWS = NP + 256
DEG_ZROWS = DEG_ROWS // NT

_mesh = functools.partial(
    plsc.VectorSubcoreMesh, core_axis_name="c", subcore_axis_name="s")


def _deg_body(dstp, zeros_h, ones_h, parts, acc_sh, dst_v, ones_v):
  c = lax.axis_index("c")
  s = lax.axis_index("s")
  w = c * NT + s
  pltpu.sync_copy(zeros_h.at[pl.ds(s * DEG_ZROWS, DEG_ZROWS)],
                  acc_sh.at[pl.ds(s * DEG_ZROWS, DEG_ZROWS)])
  pltpu.sync_copy(ones_h, ones_v)
  pltpu.sync_copy(dstp.at[w], dst_v)
  plsc.subcore_barrier()

  def step(j, carry):
    pltpu.sync_copy(ones_v, acc_sh.at[dst_v.at[j]], add=True)
    return carry

  lax.fori_loop(0, DEG_B_N, step, 0)
  plsc.subcore_barrier()
  pltpu.sync_copy(acc_sh.at[pl.ds(s * ROWS_PER_TILE, ROWS_PER_TILE)],
                  parts.at[c, pl.ds(s * ROWS_PER_TILE, ROWS_PER_TILE)])


def _deg_call(dstp, zeros_h, ones_h):
  f = pl.kernel(
      _deg_body,
      out_type=jax.ShapeDtypeStruct((NC, NP, 16), jnp.float32),
      mesh=_mesh(),
      scratch_types=[
          pltpu.VMEM_SHARED((DEG_ROWS, 16), jnp.float32),
          pltpu.VMEM((DEG_B_N, DEG_B_W), jnp.int32),
          pltpu.VMEM((DEG_B_W, 16), jnp.float32),
      ],
  )
  return f(dstp, zeros_h, ones_h)


def _agg_body(hpq, src_r, dst_r, aggq, acc_sh, src_v, dst_v, rows_0, rows_1,
              sem_g0, sem_g1):
  c = lax.axis_index("c")
  s = lax.axis_index("s")
  rows = (rows_0, rows_1)
  gsems = (sem_g0, sem_g1)

  for qq in range(NQ // NC):
    q = c * (NQ // NC) + qq
    hq = hpq.at[q]
    pltpu.sync_copy(hq.at[pl.ds(s * ROWS_PER_TILE, ROWS_PER_TILE)],
                    acc_sh.at[pl.ds(s * ROWS_PER_TILE, ROWS_PER_TILE)])
    plsc.subcore_barrier()

    def chunk(ch, carry):
      pltpu.sync_copy(src_r.at[s, pl.ds(ch * CB, CB)], src_v)
      pltpu.sync_copy(dst_r.at[s, pl.ds(ch * CB, CB)], dst_v)
      pend = pltpu.async_copy(hq.at[src_v.at[0]], rows[0], gsems[0])
      for b in range(CB):
        pend.wait()
        if b + 1 < CB:
          pend = pltpu.async_copy(hq.at[src_v.at[b + 1]], rows[(b + 1) % 2],
                                  gsems[(b + 1) % 2])
        pltpu.sync_copy(rows[b % 2], acc_sh.at[dst_v.at[b]], add=True)
      return carry

    lax.fori_loop(0, NCH, chunk, 0)
    plsc.subcore_barrier()
    pltpu.sync_copy(acc_sh.at[pl.ds(s * ROWS_PER_TILE, ROWS_PER_TILE)],
                    aggq.at[q, pl.ds(s * ROWS_PER_TILE, ROWS_PER_TILE)])
    plsc.subcore_barrier()


def _agg_call(hpq, src_r, dst_r):
  f = pl.kernel(
      _agg_body,
      out_type=jax.ShapeDtypeStruct((NQ, NP, QW), jnp.float32),
      mesh=_mesh(),
      scratch_types=[
          pltpu.VMEM_SHARED((NP, QW), jnp.float32),
          pltpu.VMEM((CB, EB_W), jnp.int32),
          pltpu.VMEM((CB, EB_W), jnp.int32),
          pltpu.VMEM((EB_W, QW), jnp.float32),
          pltpu.VMEM((EB_W, QW), jnp.float32),
          pltpu.SemaphoreType.DMA,
          pltpu.SemaphoreType.DMA,
      ],
  )
  return f(hpq, src_r, dst_r)


_PREC = lax.Precision.DEFAULT


def _dinv_body(parts, dinv):
  i = pl.program_id(0)
  d = parts[0, :, 0:1] + parts[1, :, 0:1] + 1.0
  row = i * BM + lax.broadcasted_iota(jnp.int32, (BM, 1), 0)
  dinv[...] = jnp.where(row < N, lax.rsqrt(d), 0.0)


def _dinv_call(parts):
  return pl.pallas_call(
      _dinv_body,
      grid=(GI,),
      in_specs=[pl.BlockSpec((NC, BM, 16), lambda i: (0, i, 0))],
      out_specs=pl.BlockSpec((BM, 1), lambda i: (i, 0)),
      out_shape=jax.ShapeDtypeStruct((NP, 1), jnp.float32),
  )(parts)


def _hp_body(x, w, dinv, out):
  out[0] = jnp.dot(x[...], w[...], preferred_element_type=jnp.float32,
                   precision=_PREC) * dinv[...]


def _hp_call(x, w, dinv):
  din = x.shape[1]
  return pl.pallas_call(
      _hp_body,
      grid=(GI, NQ),
      in_specs=[
          pl.BlockSpec((BM, din), lambda i, q: (i, 0)),
          pl.BlockSpec((din, QW), lambda i, q: (0, q)),
          pl.BlockSpec((BM, 1), lambda i, q: (i, 0)),
      ],
      out_specs=pl.BlockSpec((1, BM, QW), lambda i, q: (q, i, 0)),
      out_shape=jax.ShapeDtypeStruct((NQ, NP, QW), jnp.float32),
  )(x, w, dinv)


def _stats_body(aggq, dinv, sref, ssref):
  i = pl.program_id(1)
  tv = aggq[0] * dinv[...]
  col = jnp.sum(tv, axis=0, keepdims=True)
  col2 = jnp.sum(tv * tv, axis=0, keepdims=True)

  @pl.when(i == 0)
  def _():
    sref[0] = col
    ssref[0] = col2

  @pl.when(i != 0)
  def _():
    sref[0] += col
    ssref[0] += col2


def _stats_call(aggq, dinv):
  return pl.pallas_call(
      _stats_body,
      grid=(NQ, GI),
      in_specs=[
          pl.BlockSpec((1, BM, QW), lambda q, i: (q, i, 0)),
          pl.BlockSpec((BM, 1), lambda q, i: (i, 0)),
      ],
      out_specs=[
          pl.BlockSpec((1, 1, QW), lambda q, i: (q, 0, 0)),
          pl.BlockSpec((1, 1, QW), lambda q, i: (q, 0, 0)),
      ],
      out_shape=[
          jax.ShapeDtypeStruct((NQ, 1, QW), jnp.float32),
          jax.ShapeDtypeStruct((NQ, 1, QW), jnp.float32),
      ],
  )(aggq, dinv)


def _bn_relu(t, s, ss, g, be):
  m = s / N
  v = ss / N - m * m
  return jnp.maximum((t - m) * lax.rsqrt(v + 1e-5) * g + be, 0.0)


def _mid_body(agg, s1, ss1, g1, be1, w2, dinv, out):
  acc = jnp.zeros((BM, QW), jnp.float32)
  for qi in range(NQ):
    h = _bn_relu(agg[qi] * dinv[...], s1[qi], ss1[qi], g1[qi], be1[qi])
    acc += jnp.dot(h, w2[qi * QW:(qi + 1) * QW, :],
                   preferred_element_type=jnp.float32, precision=_PREC)
  out[0] = acc * dinv[...]


def _mid_call(agg, s1, ss1, g1, be1, w2, dinv):
  return pl.pallas_call(
      _mid_body,
      grid=(GI, NQ),
      in_specs=[
          pl.BlockSpec((NQ, BM, QW), lambda i, q: (0, i, 0)),
          pl.BlockSpec((NQ, 1, QW), lambda i, q: (0, 0, 0)),
          pl.BlockSpec((NQ, 1, QW), lambda i, q: (0, 0, 0)),
          pl.BlockSpec((NQ, QW), lambda i, q: (0, 0)),
          pl.BlockSpec((NQ, QW), lambda i, q: (0, 0)),
          pl.BlockSpec((H, QW), lambda i, q: (0, q)),
          pl.BlockSpec((BM, 1), lambda i, q: (i, 0)),
      ],
      out_specs=pl.BlockSpec((1, BM, QW), lambda i, q: (q, i, 0)),
      out_shape=jax.ShapeDtypeStruct((NQ, NP, QW), jnp.float32),
  )(agg, s1, ss1, g1, be1, w2, dinv)


def _fin_body(agg, s2, ss2, g2, be2, wfc, bfc, dinv, out):
  acc = jnp.zeros((BM_F, C), jnp.float32)
  for qi in range(NQ):
    h = _bn_relu(agg[qi] * dinv[...], s2[qi], ss2[qi], g2[qi], be2[qi])
    acc += jnp.dot(h, wfc[qi], preferred_element_type=jnp.float32,
                   precision=_PREC)
  out[...] = acc + bfc[...]


def _fin_call(agg, s2, ss2, g2, be2, wfcq, bfc, dinv):
  return pl.pallas_call(
      _fin_body,
      grid=(GI_F,),
      in_specs=[
          pl.BlockSpec((NQ, BM_F, QW), lambda i: (0, i, 0)),
          pl.BlockSpec((NQ, 1, QW), lambda i: (0, 0, 0)),
          pl.BlockSpec((NQ, 1, QW), lambda i: (0, 0, 0)),
          pl.BlockSpec((NQ, QW), lambda i: (0, 0)),
          pl.BlockSpec((NQ, QW), lambda i: (0, 0)),
          pl.BlockSpec((NQ, QW, C), lambda i: (0, 0, 0)),
          pl.BlockSpec((1, C), lambda i: (0, 0)),
          pl.BlockSpec((BM_F, 1), lambda i: (i, 0)),
      ],
      out_specs=pl.BlockSpec((BM_F, C), lambda i: (i, 0)),
      out_shape=jax.ShapeDtypeStruct((N, C), jnp.float32),
  )(agg, s2, ss2, g2, be2, wfcq, bfc, dinv)


def kernel(x, edge_index, W1, b1, g1, be1, W2, b2, g2, be2, Wfc, bfc):
  src = edge_index[0]
  dst = edge_index[1]
  tpad = ET_PAD - E // NT
  tid = jnp.arange(NT, dtype=jnp.int32)[:, None]
  k = jnp.arange(tpad, dtype=jnp.int32)[None, :]
  pad_src = (tid * 977 + k * 41) % N
  pad_dst = N + (tid * 31 + k) % (NP - N)
  src_r = jnp.concatenate([src.reshape(NT, E // NT), pad_src],
                          axis=1).reshape(NT, NCH * CB, EB_W)
  dst_r = jnp.concatenate([dst.reshape(NT, E // NT), pad_dst],
                          axis=1).reshape(NT, NCH * CB, EB_W)
  pad = NP + (jnp.arange(DEG_PAD, dtype=jnp.int32) % 256)
  dstp = jnp.concatenate([dst, pad]).reshape(NC * NT, DEG_B_N, DEG_B_W)
  zeros_h = jnp.zeros((DEG_ROWS, 16), jnp.float32)
  ones_h = jnp.ones((DEG_B_W, 16), jnp.float32)
  xp = jnp.pad(x, ((0, NP - N), (0, 0)))

  parts = _deg_call(dstp, zeros_h, ones_h)
  dinv = _dinv_call(parts)

  g1q = g1.reshape(NQ, QW)
  be1q = be1.reshape(NQ, QW)
  g2q = g2.reshape(NQ, QW)
  be2q = be2.reshape(NQ, QW)
  wfcq = Wfc.reshape(NQ, QW, C)
  bfc2 = bfc.reshape(1, C)

  hp1 = _hp_call(xp, W1, dinv)
  agg1 = _agg_call(hp1, src_r, dst_r)
  s1, ss1 = _stats_call(agg1, dinv)
  hp2 = _mid_call(agg1, s1, ss1, g1q, be1q, W2, dinv)
  agg2 = _agg_call(hp2, src_r, dst_r)
  s2, ss2 = _stats_call(agg2, dinv)
  out = _fin_call(agg2, s2, ss2, g2q, be2q, wfcq, bfc2, dinv)
  return out
